# baseline (speedup 1.0000x reference)
import jax
import jax.numpy as jnp
from jax.experimental import pallas as pl

_BM = 1024
_KF = 48


def _fused_body(s_ref, g_ref, c_ref, v_ref, o_ref):
    bm = s_ref.shape[0]
    p = jnp.concatenate([s_ref[...], jnp.ones((bm, 1), jnp.float32)], axis=1)
    col = jax.lax.broadcasted_iota(jnp.int32, (4, 16), 1)
    row = jax.lax.broadcasted_iota(jnp.int32, (4, 16), 0)
    t_sel = jnp.where(col % 4 == row, 1.0, 0.0).astype(jnp.float32)
    r_sel = jnp.where(col // 4 == row, 1.0, 0.0).astype(jnp.float32)
    pt = jnp.dot(p, t_sel, preferred_element_type=jnp.float32)
    pr = jnp.dot(p, r_sel, preferred_element_type=jnp.float32)
    f16 = pt * pr
    f_hi = f16.astype(jnp.bfloat16)
    f_lo = (f16 - f_hi.astype(jnp.float32)).astype(jnp.bfloat16)
    f_big = jnp.concatenate([f_hi, f_hi, f_lo], axis=1)

    s = jnp.dot(f_big, g_ref[...], preferred_element_type=jnp.float32)
    s = s + c_ref[0:1, :]
    w = jnp.exp(s)
    o_ref[...] = jnp.dot(w, v_ref[...], preferred_element_type=jnp.float32)


def _hi(x):
    return x.astype(jnp.bfloat16).astype(jnp.float32)


def kernel(means, values, covariances, conics, opacities, samples):
    del covariances
    M = samples.shape[0]
    N = means.shape[0]
    C = values.shape[1]

    A11, A12, A13, A22, A23, A33 = [conics[:, i] for i in range(6)]
    mx, my, mz = means[:, 0], means[:, 1], means[:, 2]
    bx = A11 * mx + A12 * my + A13 * mz
    by = A12 * mx + A22 * my + A23 * mz
    bz = A13 * mx + A23 * my + A33 * mz
    c = mx * bx + my * by + mz * bz
    h = [-0.5 * A11, -0.5 * A12, -0.5 * A13,
         -0.5 * A12, -0.5 * A22, -0.5 * A23,
         -0.5 * A13, -0.5 * A23, -0.5 * A33]
    hbx, hby, hbz = 0.5 * bx, 0.5 * by, 0.5 * bz
    zn = jnp.zeros((N,), jnp.float32)
    g16 = [h[0], h[1], h[2], hbx,
           h[3], h[4], h[5], hby,
           h[6], h[7], h[8], hbz,
           hbx, hby, hbz, zn]
    g_cols = ([_hi(x) for x in g16] + [x - _hi(x) for x in g16]
              + [_hi(x) for x in g16])
    g_mat = jnp.stack(g_cols, axis=1).astype(jnp.bfloat16).T
    c_mat = jnp.broadcast_to(
        (-0.5 * c + jnp.log(opacities[:, 0]))[None, :], (8, N))

    out = pl.pallas_call(
        _fused_body,
        grid=(M // _BM,),
        in_specs=[
            pl.BlockSpec((_BM, 3), lambda i: (i, 0)),
            pl.BlockSpec((_KF, N), lambda i: (0, 0)),
            pl.BlockSpec((8, N), lambda i: (0, 0)),
            pl.BlockSpec((N, C), lambda i: (0, 0)),
        ],
        out_specs=pl.BlockSpec((_BM, C), lambda i: (i, 0)),
        out_shape=jax.ShapeDtypeStruct((M, C), jnp.float32),
    )(samples, g_mat, c_mat, values)
    return out
